# baseline (speedup 1.0000x reference)
import functools

import jax
import jax.numpy as jnp
from jax import lax
from jax.experimental import pallas as pl
from jax.experimental.pallas import tpu as pltpu
from jax.experimental.pallas import tpu_sc as plsc

N = 10000
E = 320000
D = 128

NC = 2
NS = 16
NW = NC * NS

CHUNK = 128
CPT = 80
NCHUNKS = NW * CPT
E_PAD = NCHUNKS * CHUNK
GB = 8
NGROUPS = CPT // GB
NP_ = 10240
PAD_ROW = N
RPS = NP_ // NS

_MESH = plsc.VectorSubcoreMesh(core_axis_name="c", subcore_axis_name="s")


def _scalar_agg_body(gather, vals_hbm, src_hbm, dst_hbm, zeros_hbm, out_hbm,
                     sivb, divb, vbuf, acc, vsh, sem):
    cid = lax.axis_index("c")
    sid = lax.axis_index("s")
    wid = sid * NC + cid
    r0 = sid * RPS
    pltpu.sync_copy(zeros_hbm.at[pl.ds(r0, RPS)], acc.at[pl.ds(r0, RPS)])
    if gather:
        @pl.when(sid == 0)
        def _():
            pltpu.sync_copy(vals_hbm, vsh)
    else:
        for j in range(CHUNK // 16):
            vbuf[pl.ds(j * 16, 16)] = jnp.full((16,), 1.0, jnp.float32)
    plsc.subcore_barrier()

    def group(g, carry):
        c0 = wid * CPT + g * GB
        pltpu.sync_copy(dst_hbm.at[pl.ds(c0, GB)], divb)
        if gather:
            pltpu.sync_copy(src_hbm.at[pl.ds(c0, GB)], sivb)
        for j in range(GB):
            if gather:
                pltpu.async_copy(vsh.at[sivb.at[j]], vbuf, sem).wait()
            pltpu.sync_copy(vbuf, acc.at[divb.at[j]], add=True)
        return carry

    lax.fori_loop(0, NGROUPS, group, 0)
    plsc.subcore_barrier()
    pltpu.sync_copy(acc.at[pl.ds(r0, RPS)], out_hbm.at[cid].at[pl.ds(r0, RPS)])


def _make_scalar_agg(gather):
    return functools.partial(
        pl.kernel,
        out_type=jax.ShapeDtypeStruct((NC, NP_), jnp.float32),
        mesh=_MESH,
        scratch_types=[
            pltpu.VMEM((GB, CHUNK), jnp.int32),
            pltpu.VMEM((GB, CHUNK), jnp.int32),
            pltpu.VMEM((CHUNK,), jnp.float32),
            pltpu.VMEM_SHARED((NP_,), jnp.float32),
            pltpu.VMEM_SHARED((NP_,), jnp.float32),
            pltpu.SemaphoreType.DMA,
        ],
    )(functools.partial(_scalar_agg_body, gather))


_sc_scalar_agg = _make_scalar_agg(True)
_sc_degree = _make_scalar_agg(False)


@functools.partial(
    pl.kernel,
    out_type=jax.ShapeDtypeStruct((NC, NP_, D), jnp.float32),
    mesh=_MESH,
    scratch_types=[
        pltpu.VMEM((GB, CHUNK), jnp.int32),
        pltpu.VMEM((GB, CHUNK), jnp.int32),
        pltpu.VMEM((2, CHUNK, D), jnp.float32),
        pltpu.VMEM_SHARED((NP_, D), jnp.float32),
        pltpu.SemaphoreType.DMA,
        pltpu.SemaphoreType.DMA,
    ],
)
def _sc_dense_agg(hs_hbm, src_hbm, dst_hbm, zeros_hbm, out_hbm,
                  sivb, divb, rows, acc, gsem, ssem):
    cid = lax.axis_index("c")
    sid = lax.axis_index("s")
    wid = sid * NC + cid
    r0 = sid * RPS
    pltpu.sync_copy(zeros_hbm.at[pl.ds(r0, RPS)], acc.at[pl.ds(r0, RPS)])
    plsc.subcore_barrier()

    def group(g, carry):
        c0 = wid * CPT + g * GB
        pltpu.sync_copy(src_hbm.at[pl.ds(c0, GB)], sivb)
        pltpu.sync_copy(dst_hbm.at[pl.ds(c0, GB)], divb)
        g0 = pltpu.async_copy(hs_hbm.at[sivb.at[0]], rows.at[0], gsem)
        gd = [g0, None]
        sd = [None, None]
        for j in range(GB):
            b = j % 2
            gd[b].wait()
            sd[b] = pltpu.async_copy(rows.at[b], acc.at[divb.at[j]], ssem,
                                     add=True)
            if j < GB - 1:
                if j >= 1:
                    sd[1 - b].wait()
                gd[1 - b] = pltpu.async_copy(hs_hbm.at[sivb.at[j + 1]],
                                             rows.at[1 - b], gsem)
        sd[0].wait()
        sd[1].wait()
        return carry

    lax.fori_loop(0, NGROUPS, group, 0)
    plsc.subcore_barrier()
    pltpu.sync_copy(acc.at[pl.ds(r0, RPS)], out_hbm.at[cid].at[pl.ds(r0, RPS)])


RB = 1000
GRID = N // RB


def _t0_body(x_ref, w_ref, h_ref):
    h_ref[...] = jnp.dot(x_ref[...], w_ref[...],
                         preferred_element_type=jnp.float32)


def _tc_matmul(x, W1):
    return pl.pallas_call(
        _t0_body,
        grid=(GRID,),
        in_specs=[
            pl.BlockSpec((RB, D), lambda i: (i, 0)),
            pl.BlockSpec((D, D), lambda i: (0, 0)),
        ],
        out_specs=pl.BlockSpec((RB, D), lambda i: (i, 0)),
        out_shape=jax.ShapeDtypeStruct((N, D), jnp.float32),
    )(x, W1)


def _t1_body(h_ref, dsum_ref, hs_ref, dinv_ref):
    dinv = lax.rsqrt(dsum_ref[...] + 1.0)
    hs_ref[...] = h_ref[...] * dinv
    dinv_ref[...] = dinv


def _tc_scale(h, degsum):
    return pl.pallas_call(
        _t1_body,
        grid=(GRID,),
        in_specs=[
            pl.BlockSpec((RB, D), lambda i: (i, 0)),
            pl.BlockSpec((RB, 1), lambda i: (i, 0)),
        ],
        out_specs=[
            pl.BlockSpec((RB, D), lambda i: (i, 0)),
            pl.BlockSpec((RB, 1), lambda i: (i, 0)),
        ],
        out_shape=[
            jax.ShapeDtypeStruct((N, D), jnp.float32),
            jax.ShapeDtypeStruct((N, 1), jnp.float32),
        ],
    )(h, degsum)


def _t2_body(a0_ref, a1_ref, hs_ref, dinv_ref, b1_ref, w2_ref, s_ref):
    dinv = dinv_ref[...]
    o = (a0_ref[...] + a1_ref[...] + hs_ref[...]) * dinv + b1_ref[...]
    o = jnp.maximum(o, 0.0)
    s_ref[...] = jnp.dot(o, w2_ref[...], preferred_element_type=jnp.float32) * dinv


def _tc_post1(a0, a1, hs, dinv, b1, W2):
    return pl.pallas_call(
        _t2_body,
        grid=(GRID,),
        in_specs=[
            pl.BlockSpec((RB, D), lambda i: (i, 0)),
            pl.BlockSpec((RB, D), lambda i: (i, 0)),
            pl.BlockSpec((RB, D), lambda i: (i, 0)),
            pl.BlockSpec((RB, 1), lambda i: (i, 0)),
            pl.BlockSpec((1, D), lambda i: (0, 0)),
            pl.BlockSpec((D, 1), lambda i: (0, 0)),
        ],
        out_specs=pl.BlockSpec((RB, 1), lambda i: (i, 0)),
        out_shape=jax.ShapeDtypeStruct((N, 1), jnp.float32),
    )(a0, a1, hs, dinv, b1, W2)


def _t3_body(q0_ref, q1_ref, s_ref, dinv_ref, b2_ref, out_ref):
    pre = (q0_ref[...] + q1_ref[...] + s_ref[...]) * dinv_ref[...] + b2_ref[...]
    out_ref[...] = jax.nn.sigmoid(pre)


def _tc_post2(q0, q1, s, dinv, b2):
    return pl.pallas_call(
        _t3_body,
        out_shape=jax.ShapeDtypeStruct((N, 1), jnp.float32),
    )(q0, q1, s, dinv, b2)


def kernel(x, edge_index, W1, b1, W2, b2):
    npad = E_PAD - E
    src2d = jnp.concatenate(
        [edge_index[0].astype(jnp.int32),
         jnp.zeros((npad,), jnp.int32)]).reshape(NCHUNKS, CHUNK)
    dst2d = jnp.concatenate(
        [edge_index[1].astype(jnp.int32),
         jnp.full((npad,), PAD_ROW, jnp.int32)]).reshape(NCHUNKS, CHUNK)
    zeros1 = jnp.zeros((NP_,), jnp.float32)
    ones1 = jnp.ones((NP_,), jnp.float32)
    zeros2 = jnp.zeros((NP_, D), jnp.float32)

    h1 = _tc_matmul(x, W1)
    degp = _sc_degree(ones1, src2d, dst2d, zeros1)
    degsum = (degp[0, :N] + degp[1, :N]).reshape(N, 1)
    hs1, dinv = _tc_scale(h1, degsum)
    aggp = _sc_dense_agg(hs1, src2d, dst2d, zeros2)
    s = _tc_post1(aggp[0, :N], aggp[1, :N], hs1, dinv,
                  b1.reshape(1, D), W2)
    spad = jnp.concatenate([s.reshape(N), jnp.zeros((NP_ - N,), jnp.float32)])
    qp = _sc_scalar_agg(spad, src2d, dst2d, zeros1)
    out = _tc_post2(qp[0, :N].reshape(N, 1), qp[1, :N].reshape(N, 1), s, dinv,
                    b2.reshape(1, 1))
    return out
